# baseline (speedup 1.0000x reference)
import functools

import jax
import jax.numpy as jnp
from jax import lax
from jax.experimental import pallas as pl
from jax.experimental.pallas import tpu as pltpu
from jax.experimental.pallas import tpu_sc as plsc

DEL = 32
B = 16384
NC, NS, L = 2, 16, 16
NW = NC * NS
BPW = B // NW


def _sigmoid(x):
    return 1.0 / (1.0 + jnp.exp(-x))


def _body(idx_hbm, table_hbm, out_hbm, idx_v, rows_v, fout_v, sem):
    wid = lax.axis_index("s") * NC + lax.axis_index("c")
    base = wid * BPW
    pltpu.sync_copy(idx_hbm.at[pl.ds(base, BPW)], idx_v)
    pltpu.async_copy(table_hbm.at[idx_v], rows_v, sem).wait()

    even = lax.iota(jnp.int32, L) * 2
    odd = even + 1

    def row(i, carry):
        x32 = rows_v[i, :]
        w = plsc.bitcast(x32, jnp.int32)
        f_even = plsc.bitcast(w << 16, jnp.float32)
        f_odd = plsc.bitcast(w & jnp.int32(-65536), jnp.float32)
        ivec = jnp.full((L,), i, jnp.int32)
        plsc.store_scatter(fout_v, [ivec, even], _sigmoid(f_even))
        plsc.store_scatter(fout_v, [ivec, odd], _sigmoid(f_odd))
        return carry

    lax.fori_loop(0, BPW, row, 0)
    pltpu.sync_copy(fout_v, out_hbm.at[pl.ds(base, BPW)])


@jax.jit
def _emb_sigmoid(idx, table):
    mesh = plsc.VectorSubcoreMesh(core_axis_name="c", subcore_axis_name="s")
    f = functools.partial(
        pl.kernel,
        mesh=mesh,
        out_type=jax.ShapeDtypeStruct((B, DEL), jnp.float32),
        scratch_types=[
            pltpu.VMEM((BPW,), jnp.int32),
            pltpu.VMEM((BPW, DEL), jnp.bfloat16),
            pltpu.VMEM((BPW, DEL), jnp.float32),
            pltpu.SemaphoreType.DMA,
        ],
        compiler_params=pltpu.CompilerParams(
            use_tc_tiling_on_sc=False, needs_layout_passes=False
        ),
    )(_body)
    return f(idx, table.astype(jnp.bfloat16))


def kernel(idx, table):
    out = _emb_sigmoid(idx.astype(jnp.int32), table)
    return out[:, :, None]


# speedup vs baseline: 1.1824x; 1.1824x over previous
import functools

import jax
import jax.numpy as jnp
from jax import lax
from jax.experimental import pallas as pl
from jax.experimental.pallas import tpu as pltpu
from jax.experimental.pallas import tpu_sc as plsc

DEL = 32
B = 16384
NC, NS, L = 2, 16, 16
NW = NC * NS
BPW = B // NW


def _body(idx_hbm, table_hbm, out_hbm, idx_v, rows_v, sem):
    wid = lax.axis_index("s") * NC + lax.axis_index("c")
    base = wid * BPW
    pltpu.sync_copy(idx_hbm.at[pl.ds(base, BPW)], idx_v)
    pltpu.async_copy(table_hbm.at[idx_v], rows_v, sem).wait()

    def row(i, carry):
        for c in range(DEL // L):
            x = rows_v[i, pl.ds(c * L, L)]
            rows_v[i, pl.ds(c * L, L)] = 1.0 / (1.0 + jnp.exp(-x))
        return carry

    lax.fori_loop(0, BPW, row, 0)
    pltpu.sync_copy(rows_v, out_hbm.at[pl.ds(base, BPW)])


@jax.jit
def _emb_sigmoid(idx, table):
    mesh = plsc.VectorSubcoreMesh(core_axis_name="c", subcore_axis_name="s")
    f = functools.partial(
        pl.kernel,
        mesh=mesh,
        out_type=jax.ShapeDtypeStruct((B, DEL), jnp.float32),
        scratch_types=[
            pltpu.VMEM((BPW,), jnp.int32),
            pltpu.VMEM((BPW, DEL), jnp.float32),
            pltpu.SemaphoreType.DMA,
        ],
        compiler_params=pltpu.CompilerParams(use_tc_tiling_on_sc=False),
    )(_body)
    return f(idx, table)


def kernel(idx, table):
    out = _emb_sigmoid(idx.astype(jnp.int32), table)
    return out[:, :, None]
